# Initial kernel scaffold; baseline (speedup 1.0000x reference)
import functools

import jax
import jax.numpy as jnp
from jax import lax
from jax.experimental import pallas as pl
from jax.experimental.pallas import tpu as pltpu
from jax.experimental.pallas import tpu_sc as plsc

N_NODES = 100000
N_EDGES = 6400000
F = 4
LANES = 128
NC = 2
NS = 16
BLK_ROWS = 16
BLOCKS_PER_TILE = 98
EDGES_PAD = NC * NS * BLOCKS_PER_TILE * BLK_ROWS * LANES
EDGE_ROWS_PER_TILE = BLOCKS_PER_TILE * BLK_ROWS

NP = 100096
ROWS_PER_TILE_STAGE = NP // NS

assert NP % (NS * 8) == 0 and NP >= N_NODES + 1


@functools.partial(
    pl.kernel,
    mesh=plsc.VectorSubcoreMesh(core_axis_name="c", subcore_axis_name="s"),
    out_type=jax.ShapeDtypeStruct((NC * NP, F), jnp.float32),
    scratch_types=[
        pltpu.VMEM((BLK_ROWS, LANES), jnp.int32),
        pltpu.VMEM((BLK_ROWS, LANES), jnp.int32),
        pltpu.VMEM((BLK_ROWS, LANES, F), jnp.float32),
        pltpu.VMEM_SHARED((NP, F), jnp.float32),
        pltpu.VMEM_SHARED((NP, F), jnp.float32),
    ],
)
def _edge_aggregate(y_hbm, z_hbm, src_hbm, dst_hbm, out_hbm,
                    sidx, didx, msg, sh_y, sh_acc):
    c = lax.axis_index("c")
    s = lax.axis_index("s")
    r0 = s * ROWS_PER_TILE_STAGE
    pltpu.sync_copy(y_hbm.at[pl.ds(r0, ROWS_PER_TILE_STAGE)],
                    sh_y.at[pl.ds(r0, ROWS_PER_TILE_STAGE)])
    pltpu.sync_copy(z_hbm.at[pl.ds(r0, ROWS_PER_TILE_STAGE)],
                    sh_acc.at[pl.ds(r0, ROWS_PER_TILE_STAGE)])
    plsc.subcore_barrier()

    row_base = (c * NS + s) * EDGE_ROWS_PER_TILE

    def body(i, carry):
        rb = row_base + i * BLK_ROWS
        pltpu.sync_copy(src_hbm.at[pl.ds(rb, BLK_ROWS)], sidx)
        pltpu.sync_copy(dst_hbm.at[pl.ds(rb, BLK_ROWS)], didx)
        pltpu.sync_copy(sh_y.at[sidx], msg)
        pltpu.sync_copy(msg, sh_acc.at[didx], add=True)
        return carry

    lax.fori_loop(0, BLOCKS_PER_TILE, body, 0)
    plsc.subcore_barrier()
    pltpu.sync_copy(sh_acc.at[pl.ds(r0, ROWS_PER_TILE_STAGE)],
                    out_hbm.at[pl.ds(c * NP + r0, ROWS_PER_TILE_STAGE)])


def kernel(x, edge_index, W1, b1, W2, b2, W3, b3, Wr, br):
    src = edge_index[0]
    dst = edge_index[1]
    pad = EDGES_PAD - N_EDGES
    srcp = jnp.concatenate(
        [src, jnp.full((pad,), N_NODES, src.dtype)]).reshape(-1, LANES)
    dstp = jnp.concatenate(
        [dst, jnp.zeros((pad,), dst.dtype)]).reshape(-1, LANES)
    zeros_np = jnp.zeros((NP, F), jnp.float32)

    ones_y = zeros_np.at[:N_NODES].set(1.0)
    deg_parts = _edge_aggregate(ones_y, zeros_np, srcp, dstp)
    deg = deg_parts.reshape(NC, NP, F).sum(0)[:N_NODES, 0] + 1.0
    dinv = lax.rsqrt(deg)

    def layer(h, W, b):
        xw = h @ W
        f = xw.shape[1]
        y = dinv[:, None] * xw
        ypad = zeros_np.at[:N_NODES, :f].set(y)
        parts = _edge_aggregate(ypad, zeros_np, srcp, dstp)
        agg = parts.reshape(NC, NP, F).sum(0)[:N_NODES, :f]
        return jnp.tanh(dinv[:, None] * (agg + y) + b)

    h = layer(x, W1, b1)
    h = layer(h, W2, b2)
    h = layer(h, W3, b3)
    return (h @ Wr + br).sum()


# speedup vs baseline: 53.1899x; 53.1899x over previous
import functools

import jax
import jax.numpy as jnp
from jax import lax
from jax.experimental import pallas as pl
from jax.experimental.pallas import tpu as pltpu
from jax.experimental.pallas import tpu_sc as plsc

N_NODES = 100000
N_EDGES = 6400000
F = 4
LANES = 128
NC = 2
NS = 16
BLK_ROWS = 8
BLOCKS_PER_TILE = 196
EDGES_PAD = NC * NS * BLOCKS_PER_TILE * BLK_ROWS * LANES
EDGE_ROWS_PER_TILE = BLOCKS_PER_TILE * BLK_ROWS

NP = 100096
ROWS_PER_TILE_STAGE = NP // NS

assert NP % (NS * 8) == 0 and NP >= N_NODES + 1


@functools.partial(
    pl.kernel,
    mesh=plsc.VectorSubcoreMesh(core_axis_name="c", subcore_axis_name="s"),
    out_type=jax.ShapeDtypeStruct((NC * NP, F), jnp.float32),
    compiler_params=pltpu.CompilerParams(use_tc_tiling_on_sc=False),
    scratch_types=[
        pltpu.VMEM((BLK_ROWS, LANES), jnp.int32),
        pltpu.VMEM((BLK_ROWS, LANES), jnp.int32),
        pltpu.VMEM((BLK_ROWS, LANES, F), jnp.float32),
        pltpu.VMEM_SHARED((NP, F), jnp.float32),
        pltpu.VMEM_SHARED((NP, F), jnp.float32),
    ],
)
def _edge_aggregate(y_hbm, z_hbm, src_hbm, dst_hbm, out_hbm,
                    sidx, didx, msg, sh_y, sh_acc):
    c = lax.axis_index("c")
    s = lax.axis_index("s")
    r0 = s * ROWS_PER_TILE_STAGE
    pltpu.sync_copy(y_hbm.at[pl.ds(r0, ROWS_PER_TILE_STAGE)],
                    sh_y.at[pl.ds(r0, ROWS_PER_TILE_STAGE)])
    pltpu.sync_copy(z_hbm.at[pl.ds(r0, ROWS_PER_TILE_STAGE)],
                    sh_acc.at[pl.ds(r0, ROWS_PER_TILE_STAGE)])
    plsc.subcore_barrier()

    row_base = (c * NS + s) * EDGE_ROWS_PER_TILE

    def body(i, carry):
        rb = row_base + i * BLK_ROWS
        pltpu.sync_copy(src_hbm.at[pl.ds(rb, BLK_ROWS)], sidx)
        pltpu.sync_copy(dst_hbm.at[pl.ds(rb, BLK_ROWS)], didx)
        for j in range(BLK_ROWS):
            pltpu.sync_copy(sh_y.at[sidx.at[j]], msg.at[j])
            pltpu.sync_copy(msg.at[j], sh_acc.at[didx.at[j]], add=True)
        return carry

    lax.fori_loop(0, BLOCKS_PER_TILE, body, 0)
    plsc.subcore_barrier()
    pltpu.sync_copy(sh_acc.at[pl.ds(r0, ROWS_PER_TILE_STAGE)],
                    out_hbm.at[pl.ds(c * NP + r0, ROWS_PER_TILE_STAGE)])


def kernel(x, edge_index, W1, b1, W2, b2, W3, b3, Wr, br):
    src = edge_index[0]
    dst = edge_index[1]
    pad = EDGES_PAD - N_EDGES
    srcp = jnp.concatenate(
        [src, jnp.full((pad,), N_NODES, src.dtype)]).reshape(-1, LANES)
    dstp = jnp.concatenate(
        [dst, jnp.zeros((pad,), dst.dtype)]).reshape(-1, LANES)
    zeros_np = jnp.zeros((NP, F), jnp.float32)

    ones_y = zeros_np.at[:N_NODES].set(1.0)
    deg_parts = _edge_aggregate(ones_y, zeros_np, srcp, dstp)
    deg = deg_parts.reshape(NC, NP, F).sum(0)[:N_NODES, 0] + 1.0
    dinv = lax.rsqrt(deg)

    def layer(h, W, b):
        xw = h @ W
        f = xw.shape[1]
        y = dinv[:, None] * xw
        ypad = zeros_np.at[:N_NODES, :f].set(y)
        parts = _edge_aggregate(ypad, zeros_np, srcp, dstp)
        agg = parts.reshape(NC, NP, F).sum(0)[:N_NODES, :f]
        return jnp.tanh(dinv[:, None] * (agg + y) + b)

    h = layer(x, W1, b1)
    h = layer(h, W2, b2)
    h = layer(h, W3, b3)
    return (h @ Wr + br).sum()
